# baseline (speedup 1.0000x reference)
import jax
import jax.numpy as jnp
from jax import lax
from jax.experimental import pallas as pl
from jax.experimental.pallas import tpu as pltpu
from jax.experimental.pallas import tpu_sc as plsc

VOCAB_SIZE = 100000
EMB_DIM = 128
BATCH_SIZE = 4096

NUM_WORKERS = 32
B_PER_WORKER = BATCH_SIZE // NUM_WORKERS

B_TILE = 128
V_TILE = 25088
NUM_B_TILES = BATCH_SIZE // B_TILE
NUM_V_TILES = (VOCAB_SIZE + V_TILE - 1) // V_TILE


def _sc_gather_body(table_hbm, idx_hbm, out_hbm, idx_v, rows_v, sem):
    wid = lax.axis_index("s") * 2 + lax.axis_index("c")
    base = wid * B_PER_WORKER
    pltpu.sync_copy(idx_hbm.at[pl.ds(base, B_PER_WORKER)], idx_v)
    pltpu.async_copy(table_hbm.at[idx_v], rows_v, sem).wait()
    pltpu.sync_copy(rows_v, out_hbm.at[pl.ds(base, B_PER_WORKER)])


def _sc_gather(emb_table, indices):
    mesh = plsc.VectorSubcoreMesh(core_axis_name="c", subcore_axis_name="s")
    return pl.kernel(
        _sc_gather_body,
        out_type=jax.ShapeDtypeStruct((BATCH_SIZE, EMB_DIM), jnp.float32),
        mesh=mesh,
        scratch_types=[
            pltpu.VMEM((B_PER_WORKER,), jnp.int32),
            pltpu.VMEM((B_PER_WORKER, EMB_DIM), jnp.float32),
            pltpu.SemaphoreType.DMA,
        ],
    )(emb_table, indices)


def _matmul_body(emb_ref, w_ref, b_ref, out_ref):
    emb = emb_ref[...]
    w = w_ref[...]
    emb_hi = emb.astype(jnp.bfloat16)
    emb_lo = (emb - emb_hi.astype(jnp.float32)).astype(jnp.bfloat16)
    w_hi = w.astype(jnp.bfloat16)
    w_lo = (w - w_hi.astype(jnp.float32)).astype(jnp.bfloat16)
    a2 = jnp.concatenate([emb_hi, emb_lo], axis=1)
    b2 = jnp.concatenate([w_hi, w_lo], axis=1)
    acc = lax.dot_general(
        a2,
        b2,
        dimension_numbers=(((1,), (1,)), ((), ())),
        preferred_element_type=jnp.float32,
    )
    out_ref[...] = acc + b_ref[...]


def _tc_matmul(embeds, lin_w, lin_b):
    bias_2d = lin_b.reshape(1, VOCAB_SIZE)
    return pl.pallas_call(
        _matmul_body,
        out_shape=jax.ShapeDtypeStruct((BATCH_SIZE, VOCAB_SIZE), jnp.float32),
        grid=(NUM_V_TILES, NUM_B_TILES),
        in_specs=[
            pl.BlockSpec((B_TILE, EMB_DIM), lambda jv, ib: (ib, 0)),
            pl.BlockSpec((V_TILE, EMB_DIM), lambda jv, ib: (jv, 0)),
            pl.BlockSpec((1, V_TILE), lambda jv, ib: (0, jv)),
        ],
        out_specs=pl.BlockSpec((B_TILE, V_TILE), lambda jv, ib: (ib, jv)),
    )(embeds, lin_w, bias_2d)


def kernel(inputs, emb_table, lin_w, lin_b):
    embeds = _sc_gather(emb_table, inputs)
    return _tc_matmul(embeds, lin_w, lin_b)


# speedup vs baseline: 3.6193x; 3.6193x over previous
import jax
import jax.numpy as jnp
from jax import lax
from jax.experimental import pallas as pl
from jax.experimental.pallas import tpu as pltpu
from jax.experimental.pallas import tpu_sc as plsc

VOCAB_SIZE = 100000
EMB_DIM = 128
BATCH_SIZE = 4096

NUM_WORKERS = 32
B_PER_WORKER = BATCH_SIZE // NUM_WORKERS

V_TILE = 1024
NUM_V_TILES = (VOCAB_SIZE + V_TILE - 1) // V_TILE


def _sc_gather_body(table_hbm, idx_hbm, out_hbm, idx_v, rows_v, sem):
    wid = lax.axis_index("s") * 2 + lax.axis_index("c")
    base = wid * B_PER_WORKER
    pltpu.sync_copy(idx_hbm.at[pl.ds(base, B_PER_WORKER)], idx_v)
    pltpu.async_copy(table_hbm.at[idx_v], rows_v, sem).wait()
    pltpu.sync_copy(rows_v, out_hbm.at[pl.ds(base, B_PER_WORKER)])


def _sc_gather(emb_table, indices):
    mesh = plsc.VectorSubcoreMesh(core_axis_name="c", subcore_axis_name="s")
    return pl.kernel(
        _sc_gather_body,
        out_type=jax.ShapeDtypeStruct((BATCH_SIZE, EMB_DIM), jnp.float32),
        mesh=mesh,
        scratch_types=[
            pltpu.VMEM((B_PER_WORKER,), jnp.int32),
            pltpu.VMEM((B_PER_WORKER, EMB_DIM), jnp.float32),
            pltpu.SemaphoreType.DMA,
        ],
    )(emb_table, indices)


def _matmul_body(emb_ref, w_ref, b_ref, out_ref):
    emb = emb_ref[...]
    w = w_ref[...]
    emb_hi = emb.astype(jnp.bfloat16)
    emb_lo = (emb - emb_hi.astype(jnp.float32)).astype(jnp.bfloat16)
    w_hi = w.astype(jnp.bfloat16)
    w_lo = (w - w_hi.astype(jnp.float32)).astype(jnp.bfloat16)
    a2 = jnp.concatenate([w_hi, w_lo], axis=1)
    b2 = jnp.concatenate([emb_hi, emb_lo], axis=1)
    acc = lax.dot_general(
        a2,
        b2,
        dimension_numbers=(((1,), (1,)), ((), ())),
        preferred_element_type=jnp.float32,
    )
    out_ref[...] = acc + b_ref[...]


def _tc_matmul(embeds, lin_w, lin_b):
    bias_2d = lin_b.reshape(VOCAB_SIZE, 1)
    out_t = pl.pallas_call(
        _matmul_body,
        out_shape=jax.ShapeDtypeStruct((VOCAB_SIZE, BATCH_SIZE), jnp.float32),
        grid=(NUM_V_TILES,),
        in_specs=[
            pl.BlockSpec((BATCH_SIZE, EMB_DIM), lambda j: (0, 0)),
            pl.BlockSpec((V_TILE, EMB_DIM), lambda j: (j, 0)),
            pl.BlockSpec((V_TILE, 1), lambda j: (j, 0)),
        ],
        out_specs=pl.BlockSpec((V_TILE, BATCH_SIZE), lambda j: (j, 0)),
    )(embeds, lin_w, bias_2d)
    return out_t.T


def kernel(inputs, emb_table, lin_w, lin_b):
    embeds = _sc_gather(emb_table, inputs)
    return _tc_matmul(embeds, lin_w, lin_b)


# speedup vs baseline: 4.0103x; 1.1080x over previous
import jax
import jax.numpy as jnp
from jax import lax
from jax.experimental import pallas as pl
from jax.experimental.pallas import tpu as pltpu
from jax.experimental.pallas import tpu_sc as plsc

VOCAB_SIZE = 100000
EMB_DIM = 128
BATCH_SIZE = 4096

NUM_WORKERS = 32
B_PER_WORKER = BATCH_SIZE // NUM_WORKERS

V_TILE = 1024
NUM_V_TILES = (VOCAB_SIZE + V_TILE - 1) // V_TILE


def _sc_gather_body(table_hbm, idx_hbm, out_hbm, idx_v, rows_v, sem):
    wid = lax.axis_index("s") * 2 + lax.axis_index("c")
    base = wid * B_PER_WORKER
    pltpu.sync_copy(idx_hbm.at[pl.ds(base, B_PER_WORKER)], idx_v)
    pltpu.async_copy(table_hbm.at[idx_v], rows_v, sem).wait()
    pltpu.sync_copy(rows_v, out_hbm.at[pl.ds(base, B_PER_WORKER)])


def _sc_gather(emb_table, indices):
    mesh = plsc.VectorSubcoreMesh(core_axis_name="c", subcore_axis_name="s")
    return pl.kernel(
        _sc_gather_body,
        out_type=jax.ShapeDtypeStruct((BATCH_SIZE, EMB_DIM), jnp.float32),
        mesh=mesh,
        scratch_types=[
            pltpu.VMEM((B_PER_WORKER,), jnp.int32),
            pltpu.VMEM((B_PER_WORKER, EMB_DIM), jnp.float32),
            pltpu.SemaphoreType.DMA,
        ],
    )(emb_table, indices)


def _matmul_body(emb_ref, w_ref, b_ref, out_ref):
    emb = emb_ref[...]
    w = w_ref[...]
    emb_hi = emb.astype(jnp.bfloat16)
    emb_lo = (emb - emb_hi.astype(jnp.float32)).astype(jnp.bfloat16)
    w_hi = w.astype(jnp.bfloat16)
    w_lo = (w - w_hi.astype(jnp.float32)).astype(jnp.bfloat16)
    a2 = jnp.concatenate([w_hi, w_lo], axis=1)
    b2 = jnp.concatenate([emb_hi, emb_lo], axis=1)
    acc = lax.dot_general(
        a2,
        b2,
        dimension_numbers=(((1,), (1,)), ((), ())),
        preferred_element_type=jnp.float32,
    )
    out_ref[...] = acc + jnp.transpose(b_ref[...], (1, 0))


def _tc_matmul(embeds, lin_w, lin_b):
    bias_2d = lin_b.reshape(1, VOCAB_SIZE)
    out_t = pl.pallas_call(
        _matmul_body,
        out_shape=jax.ShapeDtypeStruct((VOCAB_SIZE, BATCH_SIZE), jnp.float32),
        grid=(NUM_V_TILES,),
        in_specs=[
            pl.BlockSpec((BATCH_SIZE, EMB_DIM), lambda j: (0, 0)),
            pl.BlockSpec((V_TILE, EMB_DIM), lambda j: (j, 0)),
            pl.BlockSpec((1, V_TILE), lambda j: (0, j)),
        ],
        out_specs=pl.BlockSpec((V_TILE, BATCH_SIZE), lambda j: (j, 0)),
    )(embeds, lin_w, bias_2d)
    return out_t.T


def kernel(inputs, emb_table, lin_w, lin_b):
    embeds = _sc_gather(emb_table, inputs)
    return _tc_matmul(embeds, lin_w, lin_b)
